# Initial kernel scaffold; baseline (speedup 1.0000x reference)
import jax
import jax.numpy as jnp
from jax.experimental import pallas as pl


def kernel(x, num_steps):
    raise NotImplementedError("write your pallas kernel here")


# speedup vs baseline: 3.6374x; 3.6374x over previous
import functools

import numpy as np
import jax
import jax.numpy as jnp
from jax import lax
from jax.experimental import pallas as pl
from jax.experimental.pallas import tpu as pltpu
from jax.experimental.pallas import tpu_sc as plsc

_BETA = 0.9
_TAU = 1.0 / (1.0 - _BETA)
_INV_TAU = np.float32(1.0 / _TAU)
_THRESHOLD = np.float32(1.0)
_NUM_IN = 8
_NUM_HID = 8
_NUM_OUT = 4
_STEPS = 16
_L = 16

_LCG_A = np.uint32(747796405)
_LCG_C = np.uint32(2891336453)
_SEED_M = np.uint32(2654435761)
_SEED_C = np.uint32(0x9E3779B9)
_U24_SCALE = np.float32(1.0 / (1 << 24))


def _sc_body(num_workers, chunk, x_hbm, ns_hbm, out_hbm, xv, ov, nsv):
    ncores = 2
    cid = lax.axis_index("c")
    sid = lax.axis_index("s")
    wid = sid * ncores + cid
    base = wid * chunk

    pltpu.sync_copy(x_hbm.at[pl.ds(base * _NUM_IN, chunk * _NUM_IN)], xv)
    pltpu.sync_copy(ns_hbm, nsv)

    ns_f = nsv[...].astype(jnp.float32)
    iota = lax.iota(jnp.int32, _L)
    ones = jnp.full((_L,), 1.0, jnp.float32)
    zeros = jnp.full((_L,), 0.0, jnp.float32)
    cols = [jnp.full((_L,), c, jnp.int32) for c in range(max(_NUM_IN, _NUM_OUT))]

    gidx = jnp.full((_L,), base, jnp.int32) + iota
    seeds0 = []
    for c in range(_NUM_IN):
        s = (gidx * np.int32(_NUM_IN) + np.int32(c)).astype(jnp.uint32)
        s = s * _SEED_M + _SEED_C
        s = s ^ (s >> np.uint32(16))
        s = s * _LCG_A + _LCG_C
        seeds0.append(s)

    num_groups = chunk // _L

    def group_body(g, carry):
        rng, row = carry
        row_in = row * np.int32(_NUM_IN)
        row_out = row * np.int32(_NUM_OUT)
        xs = [plsc.load_gather(xv, [row_in + cols[c]]) for c in range(_NUM_IN)]

        def step(carry):
            rng, vh, vo, acc, t_f = carry
            active = jnp.where(t_f < ns_f, ones, zeros)
            spikes_in = []
            new_rng = []
            for c in range(_NUM_IN):
                s = rng[c] * _LCG_A + _LCG_C
                w = s ^ (s >> np.uint32(16))
                u = plsc.bitcast(w >> np.uint32(8), jnp.int32).astype(jnp.float32)
                u = u * _U24_SCALE
                spikes_in.append(jnp.where(u < xs[c], ones, zeros))
                new_rng.append(s)
            hs = []
            new_vh = []
            for h in range(_NUM_HID):
                agg = spikes_in[h] - spikes_in[(h + 3) % _NUM_IN]
                v_new = vh[h] + (agg - vh[h]) * _INV_TAU
                spk = jnp.where(v_new >= _THRESHOLD, ones, zeros)
                new_vh.append(v_new * (ones - spk))
                hs.append(spk)
            new_vo = []
            new_acc = []
            for o in range(_NUM_OUT):
                agg = (hs[2 * o] + hs[2 * o + 1]
                       - hs[(2 * o + 4) % _NUM_HID] - hs[(2 * o + 5) % _NUM_HID])
                v_new = vo[o] + (agg - vo[o]) * _INV_TAU
                spk = jnp.where(v_new >= _THRESHOLD, ones, zeros)
                new_vo.append(v_new * (ones - spk))
                new_acc.append(acc[o] + spk * active)
            return (tuple(new_rng), tuple(new_vh), tuple(new_vo),
                    tuple(new_acc), t_f + 1.0)

        init = (rng, (zeros,) * _NUM_HID, (zeros,) * _NUM_OUT,
                (zeros,) * _NUM_OUT, zeros)
        rng, _, _, acc, _ = lax.fori_loop(
            0, _STEPS, lambda t, c: step(c), init)
        for o in range(_NUM_OUT):
            plsc.store_scatter(ov, [row_out + cols[o]], acc[o])
        return (rng, row + np.int32(_L))

    lax.fori_loop(0, num_groups, group_body, (tuple(seeds0), iota))

    pltpu.sync_copy(ov, out_hbm.at[pl.ds(base * _NUM_OUT, chunk * _NUM_OUT)])


def kernel(x, num_steps):
    batch = x.shape[0]
    num_workers = 32
    assert batch % (num_workers * _L) == 0
    chunk = batch // num_workers
    ns_arr = jnp.full((_L,), num_steps, dtype=jnp.int32)

    mesh = plsc.VectorSubcoreMesh(core_axis_name="c", subcore_axis_name="s")
    run = pl.kernel(
        functools.partial(_sc_body, num_workers, chunk),
        out_type=jax.ShapeDtypeStruct((batch * _NUM_OUT,), jnp.float32),
        mesh=mesh,
        compiler_params=pltpu.CompilerParams(needs_layout_passes=False),
        scratch_types=[
            pltpu.VMEM((chunk * _NUM_IN,), jnp.float32),
            pltpu.VMEM((chunk * _NUM_OUT,), jnp.float32),
            pltpu.VMEM((_L,), jnp.int32),
        ],
    )
    out_flat = run(x.reshape(batch * _NUM_IN), ns_arr)
    return out_flat.reshape(batch, _NUM_OUT)


# speedup vs baseline: 3.9074x; 1.0742x over previous
import functools

import numpy as np
import jax
import jax.numpy as jnp
from jax import lax
from jax.experimental import pallas as pl
from jax.experimental.pallas import tpu as pltpu
from jax.experimental.pallas import tpu_sc as plsc

_BETA = 0.9
_TAU = 1.0 / (1.0 - _BETA)
_INV_TAU = np.float32(1.0 / _TAU)
_THRESHOLD = np.float32(1.0)
_NUM_IN = 8
_NUM_HID = 8
_NUM_OUT = 4
_STEPS = 16
_L = 16

_LCG_A = np.uint32(747796405)
_LCG_C = np.uint32(2891336453)
_SEED_M = np.uint32(2654435761)
_SEED_C = np.uint32(0x9E3779B9)


def _sc_body(num_workers, chunk, x_hbm, ns_hbm, out_hbm, xv, ov, nsv):
    ncores = 2
    cid = lax.axis_index("c")
    sid = lax.axis_index("s")
    wid = sid * ncores + cid
    base = wid * chunk

    pltpu.sync_copy(x_hbm.at[pl.ds(base * _NUM_IN, chunk * _NUM_IN)], xv)
    pltpu.sync_copy(ns_hbm, nsv)

    iota = lax.iota(jnp.int32, _L)
    ones = jnp.full((_L,), 1.0, jnp.float32)
    zeros = jnp.full((_L,), 0.0, jnp.float32)
    cols = [jnp.full((_L,), c, jnp.int32) for c in range(max(_NUM_IN, _NUM_OUT))]

    gidx = jnp.full((_L,), base, jnp.int32) + iota
    seeds0 = []
    for c in range(_NUM_IN):
        s = (gidx * np.int32(_NUM_IN) + np.int32(c)).astype(jnp.uint32)
        s = s * _SEED_M + _SEED_C
        s = s ^ (s >> np.uint32(16))
        s = s * _LCG_A + _LCG_C
        seeds0.append(s)

    num_groups = chunk // _L

    ns_i = nsv[...]

    def group_body(g, carry):
        rng, row = carry
        row_in = row * np.int32(_NUM_IN)
        row_out = row * np.int32(_NUM_OUT)
        thr = [
            (plsc.load_gather(xv, [row_in + cols[c]]) * np.float32(1 << 24))
            .astype(jnp.int32)
            for c in range(_NUM_IN)
        ]

        rng = list(rng)
        vh = [zeros] * _NUM_HID
        vo = [zeros] * _NUM_OUT
        acc = [zeros] * _NUM_OUT
        for t in range(_STEPS):
            active = jnp.where(jnp.full((_L,), t, jnp.int32) < ns_i, ones, zeros)
            spikes_in = []
            for c in range(_NUM_IN):
                s = rng[c] * _LCG_A + _LCG_C
                u = plsc.bitcast(s >> np.uint32(8), jnp.int32)
                spikes_in.append(jnp.where(u < thr[c], ones, zeros))
                rng[c] = s
            hs = []
            for h in range(_NUM_HID):
                agg = spikes_in[h] - spikes_in[(h + 3) % _NUM_IN]
                v_new = vh[h] + (agg - vh[h]) * _INV_TAU
                fired = v_new >= _THRESHOLD
                hs.append(jnp.where(fired, ones, zeros))
                vh[h] = jnp.where(fired, zeros, v_new)
            for o in range(_NUM_OUT):
                agg = (hs[2 * o] + hs[2 * o + 1]
                       - hs[(2 * o + 4) % _NUM_HID] - hs[(2 * o + 5) % _NUM_HID])
                v_new = vo[o] + (agg - vo[o]) * _INV_TAU
                fired = v_new >= _THRESHOLD
                vo[o] = jnp.where(fired, zeros, v_new)
                acc[o] = acc[o] + jnp.where(fired, active, zeros)
        for o in range(_NUM_OUT):
            plsc.store_scatter(ov, [row_out + cols[o]], acc[o])
        return (tuple(rng), row + np.int32(_L))

    lax.fori_loop(0, num_groups, group_body, (tuple(seeds0), iota))

    pltpu.sync_copy(ov, out_hbm.at[pl.ds(base * _NUM_OUT, chunk * _NUM_OUT)])


def kernel(x, num_steps):
    batch = x.shape[0]
    num_workers = 32
    assert batch % (num_workers * _L) == 0
    chunk = batch // num_workers
    ns_arr = jnp.full((_L,), num_steps, dtype=jnp.int32)

    mesh = plsc.VectorSubcoreMesh(core_axis_name="c", subcore_axis_name="s")
    run = pl.kernel(
        functools.partial(_sc_body, num_workers, chunk),
        out_type=jax.ShapeDtypeStruct((batch * _NUM_OUT,), jnp.float32),
        mesh=mesh,
        compiler_params=pltpu.CompilerParams(needs_layout_passes=False),
        scratch_types=[
            pltpu.VMEM((chunk * _NUM_IN,), jnp.float32),
            pltpu.VMEM((chunk * _NUM_OUT,), jnp.float32),
            pltpu.VMEM((_L,), jnp.int32),
        ],
    )
    out_flat = run(x.reshape(batch * _NUM_IN), ns_arr)
    return out_flat.reshape(batch, _NUM_OUT)


# speedup vs baseline: 6.9722x; 1.7844x over previous
import functools

import numpy as np
import jax
import jax.numpy as jnp
from jax import lax
from jax.experimental import pallas as pl
from jax.experimental.pallas import tpu as pltpu
from jax.experimental.pallas import tpu_sc as plsc

_BETA = 0.9
_TAU = 1.0 / (1.0 - _BETA)
_INV_TAU = np.float32(1.0 / _TAU)
_THRESHOLD = np.float32(1.0)
_NUM_IN = 8
_NUM_HID = 8
_NUM_OUT = 4
_STEPS = 16
_L = 16

_LCG_A = np.uint32(747796405)
_LCG_C = np.uint32(2891336453)
_SEED_M = np.uint32(2654435761)
_SEED_C = np.uint32(0x9E3779B9)


def _sc_body(num_workers, chunk, x_hbm, ns_hbm, out_hbm, xv, ov, nsv, sem):
    ncores = 2
    cid = lax.axis_index("c")
    sid = lax.axis_index("s")
    wid = sid * ncores + cid
    base = wid * chunk

    copies = [
        pltpu.make_async_copy(
            x_hbm.at[c, pl.ds(base, chunk)],
            xv.at[pl.ds(c * chunk, chunk)],
            sem,
        )
        for c in range(_NUM_IN)
    ]
    for cp in copies:
        cp.start()
    pltpu.sync_copy(ns_hbm, nsv)
    for cp in copies:
        cp.wait()

    iota = lax.iota(jnp.int32, _L)
    ones = jnp.full((_L,), 1.0, jnp.float32)
    zeros = jnp.full((_L,), 0.0, jnp.float32)
    ns_i = nsv[...]

    gidx = jnp.full((_L,), base, jnp.int32) + iota
    seeds0 = []
    for c in range(_NUM_IN):
        s = (gidx * np.int32(_NUM_IN) + np.int32(c)).astype(jnp.uint32)
        s = s * _SEED_M + _SEED_C
        s = s ^ (s >> np.uint32(16))
        s = s * _LCG_A + _LCG_C
        seeds0.append(s)

    num_groups = chunk // _L

    def group_body(g, rng):
        off = g * np.int32(_L)
        thr = [
            (xv[pl.ds(c * chunk + off, _L)] * np.float32(1 << 24))
            .astype(jnp.int32)
            for c in range(_NUM_IN)
        ]

        rng = list(rng)
        vh = [zeros] * _NUM_HID
        vo = [zeros] * _NUM_OUT
        acc = [zeros] * _NUM_OUT
        for t in range(_STEPS):
            active = jnp.where(jnp.full((_L,), t, jnp.int32) < ns_i, ones, zeros)
            spikes_in = []
            for c in range(_NUM_IN):
                s = rng[c] * _LCG_A + _LCG_C
                u = plsc.bitcast(s >> np.uint32(8), jnp.int32)
                spikes_in.append(jnp.where(u < thr[c], ones, zeros))
                rng[c] = s
            hs = []
            for h in range(_NUM_HID):
                agg = spikes_in[h] - spikes_in[(h + 3) % _NUM_IN]
                v_new = vh[h] + (agg - vh[h]) * _INV_TAU
                fired = v_new >= _THRESHOLD
                hs.append(jnp.where(fired, ones, zeros))
                vh[h] = jnp.where(fired, zeros, v_new)
            for o in range(_NUM_OUT):
                agg = (hs[2 * o] + hs[2 * o + 1]
                       - hs[(2 * o + 4) % _NUM_HID] - hs[(2 * o + 5) % _NUM_HID])
                v_new = vo[o] + (agg - vo[o]) * _INV_TAU
                fired = v_new >= _THRESHOLD
                vo[o] = jnp.where(fired, zeros, v_new)
                acc[o] = acc[o] + jnp.where(fired, active, zeros)
        for o in range(_NUM_OUT):
            ov[pl.ds(o * chunk + off, _L)] = acc[o]
        return tuple(rng)

    lax.fori_loop(0, num_groups, group_body, tuple(seeds0))

    out_copies = [
        pltpu.make_async_copy(
            ov.at[pl.ds(o * chunk, chunk)],
            out_hbm.at[o, pl.ds(base, chunk)],
            sem,
        )
        for o in range(_NUM_OUT)
    ]
    for cp in out_copies:
        cp.start()
    for cp in out_copies:
        cp.wait()


def kernel(x, num_steps):
    batch = x.shape[0]
    num_workers = 32
    assert batch % (num_workers * _L) == 0
    chunk = batch // num_workers
    ns_arr = jnp.full((_L,), num_steps, dtype=jnp.int32)

    mesh = plsc.VectorSubcoreMesh(core_axis_name="c", subcore_axis_name="s")
    run = pl.kernel(
        functools.partial(_sc_body, num_workers, chunk),
        out_type=jax.ShapeDtypeStruct((_NUM_OUT, batch), jnp.float32),
        mesh=mesh,
        compiler_params=pltpu.CompilerParams(needs_layout_passes=False),
        scratch_types=[
            pltpu.VMEM((chunk * _NUM_IN,), jnp.float32),
            pltpu.VMEM((chunk * _NUM_OUT,), jnp.float32),
            pltpu.VMEM((_L,), jnp.int32),
            pltpu.SemaphoreType.DMA,
        ],
    )
    out_t = run(x.T, ns_arr)
    return out_t.T


# speedup vs baseline: 8.1354x; 1.1668x over previous
import functools

import numpy as np
import jax
import jax.numpy as jnp
from jax import lax
from jax.experimental import pallas as pl
from jax.experimental.pallas import tpu as pltpu
from jax.experimental.pallas import tpu_sc as plsc

_BETA = 0.9
_TAU = 1.0 / (1.0 - _BETA)
_INV_TAU = np.float32(1.0 / _TAU)
_THRESHOLD = np.float32(1.0)
_NUM_IN = 8
_NUM_HID = 8
_NUM_OUT = 4
_STEPS = 16
_L = 16

_LCG_A = np.uint32(747796405)
_LCG_C = np.uint32(2891336453)
_SEED_M = np.uint32(2654435761)
_SEED_C = np.uint32(0x9E3779B9)


def _sc_body(num_workers, chunk, sc_base, x_hbm, ns_hbm, out_hbm, xv, ov, nsv, sem):
    ncores = 2
    cid = lax.axis_index("c")
    sid = lax.axis_index("s")
    wid = sid * ncores + cid
    base = sc_base + wid * chunk

    copies = [
        pltpu.make_async_copy(
            x_hbm.at[c, pl.ds(base, chunk)],
            xv.at[pl.ds(c * chunk, chunk)],
            sem,
        )
        for c in range(_NUM_IN)
    ]
    for cp in copies:
        cp.start()
    pltpu.sync_copy(ns_hbm, nsv)
    for cp in copies:
        cp.wait()

    iota = lax.iota(jnp.int32, _L)
    ones = jnp.full((_L,), 1.0, jnp.float32)
    zeros = jnp.full((_L,), 0.0, jnp.float32)
    ns_i = nsv[...]

    gidx = jnp.full((_L,), base, jnp.int32) + iota
    seeds0 = []
    for c in range(_NUM_IN):
        s = (gidx * np.int32(_NUM_IN) + np.int32(c)).astype(jnp.uint32)
        s = s * _SEED_M + _SEED_C
        s = s ^ (s >> np.uint32(16))
        s = s * _LCG_A + _LCG_C
        seeds0.append(s)

    num_groups = chunk // _L

    def group_body(g, rng):
        off = g * np.int32(_L)
        thr = [
            (xv[pl.ds(c * chunk + off, _L)] * np.float32(1 << 24))
            .astype(jnp.int32)
            for c in range(_NUM_IN)
        ]

        rng = list(rng)
        vh = [zeros] * _NUM_HID
        vo = [zeros] * _NUM_OUT
        acc = [zeros] * _NUM_OUT
        for t in range(_STEPS):
            active = jnp.where(jnp.full((_L,), t, jnp.int32) < ns_i, ones, zeros)
            spikes_in = []
            for c in range(_NUM_IN):
                s = rng[c] * _LCG_A + _LCG_C
                u = plsc.bitcast(s >> np.uint32(8), jnp.int32)
                spikes_in.append(jnp.where(u < thr[c], ones, zeros))
                rng[c] = s
            hs = []
            for h in range(_NUM_HID):
                agg = spikes_in[h] - spikes_in[(h + 3) % _NUM_IN]
                v_new = vh[h] + (agg - vh[h]) * _INV_TAU
                fired = v_new >= _THRESHOLD
                hs.append(jnp.where(fired, ones, zeros))
                vh[h] = jnp.where(fired, zeros, v_new)
            for o in range(_NUM_OUT):
                agg = (hs[2 * o] + hs[2 * o + 1]
                       - hs[(2 * o + 4) % _NUM_HID] - hs[(2 * o + 5) % _NUM_HID])
                v_new = vo[o] + (agg - vo[o]) * _INV_TAU
                fired = v_new >= _THRESHOLD
                vo[o] = jnp.where(fired, zeros, v_new)
                acc[o] = acc[o] + jnp.where(fired, active, zeros)
        for o in range(_NUM_OUT):
            ov[pl.ds(o * chunk + off, _L)] = acc[o]
        return tuple(rng)

    lax.fori_loop(0, num_groups, group_body, tuple(seeds0))

    out_copies = [
        pltpu.make_async_copy(
            ov.at[pl.ds(o * chunk, chunk)],
            out_hbm.at[o, pl.ds(base - sc_base, chunk)],
            sem,
        )
        for o in range(_NUM_OUT)
    ]
    for cp in out_copies:
        cp.start()
    for cp in out_copies:
        cp.wait()


def _tc_body(ns_ref, x_ref, out_ref):
    bt = x_ref.shape[1]
    pltpu.prng_seed(0x5CBA17)
    thr = (x_ref[...] * np.float32(1 << 24)).astype(jnp.int32)
    ns = ns_ref[0]
    zero = np.float32(0.0)
    one = np.float32(1.0)
    vh = jnp.zeros((_NUM_HID, bt), jnp.float32)
    vo = jnp.zeros((_NUM_HID, bt), jnp.float32)
    acc = jnp.zeros((_NUM_HID, bt), jnp.float32)
    for t in range(_STEPS):
        bits = pltpu.prng_random_bits((_NUM_IN, bt)).astype(jnp.uint32)
        u = (bits >> np.uint32(8)).astype(jnp.int32)
        spikes = jnp.where(u < thr, one, zero)
        aggh = spikes - jnp.concatenate([spikes[3:], spikes[:3]], axis=0)
        v_new = vh + (aggh - vh) * _INV_TAU
        firedh = v_new >= _THRESHOLD
        hsp = jnp.where(firedh, one, zero)
        vh = jnp.where(firedh, zero, v_new)
        q = hsp + jnp.concatenate([hsp[1:], hsp[:1]], axis=0)
        aggo = q - jnp.concatenate([q[4:], q[:4]], axis=0)
        v_new_o = vo + (aggo - vo) * _INV_TAU
        firedo = v_new_o >= _THRESHOLD
        vo = jnp.where(firedo, zero, v_new_o)
        active = jnp.where(t < ns, one, zero)
        acc = acc + jnp.where(firedo, active, zero)
    for o in range(_NUM_OUT):
        out_ref[o, :] = acc[2 * o, :]


def kernel(x, num_steps):
    batch = x.shape[0]
    num_workers = 32
    b_tc = batch // 2
    b_sc = batch - b_tc
    assert b_sc % (num_workers * _L) == 0 and b_tc % 128 == 0
    chunk = b_sc // num_workers
    ns_arr = jnp.full((_L,), num_steps, dtype=jnp.int32)
    xt = x.T

    mesh = plsc.VectorSubcoreMesh(core_axis_name="c", subcore_axis_name="s")
    run_sc = pl.kernel(
        functools.partial(_sc_body, num_workers, chunk, b_tc),
        out_type=jax.ShapeDtypeStruct((_NUM_OUT, b_sc), jnp.float32),
        mesh=mesh,
        compiler_params=pltpu.CompilerParams(needs_layout_passes=False),
        scratch_types=[
            pltpu.VMEM((chunk * _NUM_IN,), jnp.float32),
            pltpu.VMEM((chunk * _NUM_OUT,), jnp.float32),
            pltpu.VMEM((_L,), jnp.int32),
            pltpu.SemaphoreType.DMA,
        ],
    )
    sc_out = run_sc(xt, ns_arr)

    tc_out = pl.pallas_call(
        _tc_body,
        out_shape=jax.ShapeDtypeStruct((_NUM_OUT, b_tc), jnp.float32),
        grid=(1,),
        in_specs=[
            pl.BlockSpec(memory_space=pltpu.SMEM),
            pl.BlockSpec((_NUM_IN, b_tc), lambda i: (0, 0)),
        ],
        out_specs=pl.BlockSpec((_NUM_OUT, b_tc), lambda i: (0, 0)),
    )(ns_arr, xt)

    out_t = jnp.concatenate([tc_out, sc_out], axis=1)
    return out_t.T


# speedup vs baseline: 9.1923x; 1.1299x over previous
import functools

import numpy as np
import jax
import jax.numpy as jnp
from jax import lax
from jax.experimental import pallas as pl
from jax.experimental.pallas import tpu as pltpu
from jax.experimental.pallas import tpu_sc as plsc

_BETA = 0.9
_TAU = 1.0 / (1.0 - _BETA)
_INV_TAU = np.float32(1.0 / _TAU)
_THRESHOLD = np.float32(1.0)
_NUM_IN = 8
_NUM_HID = 8
_NUM_OUT = 4
_STEPS = 16
_L = 16

_LCG_A = np.uint32(747796405)
_LCG_C = np.uint32(2891336453)
_SEED_M = np.uint32(2654435761)
_SEED_C = np.uint32(0x9E3779B9)


def _sc_body(num_workers, chunk, sc_base, x_hbm, ns_hbm, out_hbm, xv, ov, nsv, sem):
    ncores = 2
    cid = lax.axis_index("c")
    sid = lax.axis_index("s")
    wid = sid * ncores + cid
    base = sc_base + wid * chunk

    copies = [
        pltpu.make_async_copy(
            x_hbm.at[c, pl.ds(base, chunk)],
            xv.at[pl.ds(c * chunk, chunk)],
            sem,
        )
        for c in range(_NUM_IN)
    ]
    for cp in copies:
        cp.start()
    pltpu.sync_copy(ns_hbm, nsv)
    for cp in copies:
        cp.wait()

    iota = lax.iota(jnp.int32, _L)
    ones = jnp.full((_L,), 1.0, jnp.float32)
    zeros = jnp.full((_L,), 0.0, jnp.float32)
    ns_i = nsv[...]

    gidx = jnp.full((_L,), base, jnp.int32) + iota
    seeds0 = []
    for c in range(_NUM_IN):
        s = (gidx * np.int32(_NUM_IN) + np.int32(c)).astype(jnp.uint32)
        s = s * _SEED_M + _SEED_C
        s = s ^ (s >> np.uint32(16))
        s = s * _LCG_A + _LCG_C
        seeds0.append(s)

    num_groups = chunk // _L

    def group_body(g, rng):
        off = g * np.int32(_L)
        thr = [
            (xv[pl.ds(c * chunk + off, _L)] * np.float32(1 << 24))
            .astype(jnp.int32)
            for c in range(_NUM_IN)
        ]

        rng = list(rng)
        vh = [zeros] * _NUM_HID
        vo = [zeros] * _NUM_OUT
        acc = [zeros] * _NUM_OUT
        for t in range(_STEPS):
            active = jnp.where(jnp.full((_L,), t, jnp.int32) < ns_i, ones, zeros)
            spikes_in = []
            for c in range(_NUM_IN):
                s = rng[c] * _LCG_A + _LCG_C
                u = plsc.bitcast(s >> np.uint32(8), jnp.int32)
                spikes_in.append(jnp.where(u < thr[c], ones, zeros))
                rng[c] = s
            hs = []
            for h in range(_NUM_HID):
                agg = spikes_in[h] - spikes_in[(h + 3) % _NUM_IN]
                v_new = vh[h] + (agg - vh[h]) * _INV_TAU
                fired = v_new >= _THRESHOLD
                hs.append(jnp.where(fired, ones, zeros))
                vh[h] = jnp.where(fired, zeros, v_new)
            for o in range(_NUM_OUT):
                agg = (hs[2 * o] + hs[2 * o + 1]
                       - hs[(2 * o + 4) % _NUM_HID] - hs[(2 * o + 5) % _NUM_HID])
                v_new = vo[o] + (agg - vo[o]) * _INV_TAU
                fired = v_new >= _THRESHOLD
                vo[o] = jnp.where(fired, zeros, v_new)
                acc[o] = acc[o] + jnp.where(fired, active, zeros)
        for o in range(_NUM_OUT):
            ov[pl.ds(o * chunk + off, _L)] = acc[o]
        return tuple(rng)

    lax.fori_loop(0, num_groups, group_body, tuple(seeds0))

    out_copies = [
        pltpu.make_async_copy(
            ov.at[pl.ds(o * chunk, chunk)],
            out_hbm.at[o, pl.ds(base - sc_base, chunk)],
            sem,
        )
        for o in range(_NUM_OUT)
    ]
    for cp in out_copies:
        cp.start()
    for cp in out_copies:
        cp.wait()


def _tc_body(ns_ref, x_ref, out_ref):
    bt = x_ref.shape[1]
    pltpu.prng_seed(0x5CBA17)
    thr = (x_ref[...] * np.float32(1 << 24)).astype(jnp.int32)
    ns = ns_ref[0]
    zero = np.float32(0.0)
    one = np.float32(1.0)
    vh = jnp.zeros((_NUM_HID, bt), jnp.float32)
    vo = jnp.zeros((_NUM_HID, bt), jnp.float32)
    acc = jnp.zeros((_NUM_HID, bt), jnp.float32)
    for t in range(_STEPS):
        bits = pltpu.prng_random_bits((_NUM_IN, bt)).astype(jnp.uint32)
        u = (bits >> np.uint32(8)).astype(jnp.int32)
        spikes = jnp.where(u < thr, one, zero)
        aggh = spikes - jnp.concatenate([spikes[3:], spikes[:3]], axis=0)
        v_new = vh + (aggh - vh) * _INV_TAU
        firedh = v_new >= _THRESHOLD
        hsp = jnp.where(firedh, one, zero)
        vh = jnp.where(firedh, zero, v_new)
        q = hsp + jnp.concatenate([hsp[1:], hsp[:1]], axis=0)
        aggo = q - jnp.concatenate([q[4:], q[:4]], axis=0)
        v_new_o = vo + (aggo - vo) * _INV_TAU
        firedo = v_new_o >= _THRESHOLD
        vo = jnp.where(firedo, zero, v_new_o)
        active = jnp.where(t < ns, one, zero)
        acc = acc + jnp.where(firedo, active, zero)
    for o in range(_NUM_OUT):
        out_ref[o, :] = acc[2 * o, :]


def kernel(x, num_steps):
    batch = x.shape[0]
    num_workers = 32
    b_tc = (batch * 3) // 4
    b_sc = batch - b_tc
    assert b_sc % (num_workers * _L) == 0 and b_tc % 128 == 0
    chunk = b_sc // num_workers
    ns_arr = jnp.full((_L,), num_steps, dtype=jnp.int32)
    xt = x.T

    mesh = plsc.VectorSubcoreMesh(core_axis_name="c", subcore_axis_name="s")
    run_sc = pl.kernel(
        functools.partial(_sc_body, num_workers, chunk, b_tc),
        out_type=jax.ShapeDtypeStruct((_NUM_OUT, b_sc), jnp.float32),
        mesh=mesh,
        compiler_params=pltpu.CompilerParams(needs_layout_passes=False),
        scratch_types=[
            pltpu.VMEM((chunk * _NUM_IN,), jnp.float32),
            pltpu.VMEM((chunk * _NUM_OUT,), jnp.float32),
            pltpu.VMEM((_L,), jnp.int32),
            pltpu.SemaphoreType.DMA,
        ],
    )
    sc_out = run_sc(xt, ns_arr)

    tc_out = pl.pallas_call(
        _tc_body,
        out_shape=jax.ShapeDtypeStruct((_NUM_OUT, b_tc), jnp.float32),
        grid=(1,),
        in_specs=[
            pl.BlockSpec(memory_space=pltpu.SMEM),
            pl.BlockSpec((_NUM_IN, b_tc), lambda i: (0, 0)),
        ],
        out_specs=pl.BlockSpec((_NUM_OUT, b_tc), lambda i: (0, 0)),
    )(ns_arr, xt)

    out_t = jnp.concatenate([tc_out, sc_out], axis=1)
    return out_t.T


# speedup vs baseline: 9.2475x; 1.0060x over previous
import functools

import numpy as np
import jax
import jax.numpy as jnp
from jax import lax
from jax.experimental import pallas as pl
from jax.experimental.pallas import tpu as pltpu
from jax.experimental.pallas import tpu_sc as plsc

_BETA = 0.9
_TAU = 1.0 / (1.0 - _BETA)
_INV_TAU = np.float32(1.0 / _TAU)
_THRESHOLD = np.float32(1.0)
_NUM_IN = 8
_NUM_HID = 8
_NUM_OUT = 4
_STEPS = 16
_L = 16

_LCG_A = np.uint32(747796405)
_LCG_C = np.uint32(2891336453)
_SEED_M = np.uint32(2654435761)
_SEED_C = np.uint32(0x9E3779B9)


def _sc_body(num_workers, chunk, sc_base, x_hbm, ns_hbm, out_hbm, xv, ov, nsv, sem):
    ncores = 2
    cid = lax.axis_index("c")
    sid = lax.axis_index("s")
    wid = sid * ncores + cid
    base = sc_base + wid * chunk

    copies = [
        pltpu.make_async_copy(
            x_hbm.at[c, pl.ds(base, chunk)],
            xv.at[pl.ds(c * chunk, chunk)],
            sem,
        )
        for c in range(_NUM_IN)
    ]
    for cp in copies:
        cp.start()
    pltpu.sync_copy(ns_hbm, nsv)
    for cp in copies:
        cp.wait()

    iota = lax.iota(jnp.int32, _L)
    ones = jnp.full((_L,), 1.0, jnp.float32)
    zeros = jnp.full((_L,), 0.0, jnp.float32)
    ns_i = nsv[...]

    actives = [
        jnp.where(jnp.full((_L,), t, jnp.int32) < ns_i, ones, zeros)
        for t in range(_STEPS)
    ]

    gidx = jnp.full((_L,), base, jnp.int32) + iota
    seeds0 = []
    for p in range(_NUM_IN // 2):
        s = (gidx * np.int32(_NUM_IN // 2) + np.int32(p)).astype(jnp.uint32)
        s = s * _SEED_M + _SEED_C
        s = s ^ (s >> np.uint32(16))
        s = s * _LCG_A + _LCG_C
        seeds0.append(s)

    num_groups = chunk // _L

    def group_body(g, rng):
        off = g * np.int32(_L)
        thr = [
            (xv[pl.ds(c * chunk + off, _L)] * np.float32(1 << 16))
            .astype(jnp.int32)
            for c in range(_NUM_IN)
        ]

        rng = list(rng)
        vh = [zeros] * _NUM_HID
        vo = [zeros] * _NUM_OUT
        acc = [zeros] * _NUM_OUT
        for t in range(_STEPS):
            active = actives[t]
            spikes_in = [None] * _NUM_IN
            for p in range(_NUM_IN // 2):
                s = rng[p] * _LCG_A + _LCG_C
                u_hi = plsc.bitcast(s >> np.uint32(16), jnp.int32)
                u_lo = plsc.bitcast(s & np.uint32(0xFFFF), jnp.int32)
                spikes_in[p] = jnp.where(u_hi < thr[p], ones, zeros)
                spikes_in[p + 4] = jnp.where(u_lo < thr[p + 4], ones, zeros)
                rng[p] = s
            hs = []
            for h in range(_NUM_HID):
                agg = spikes_in[h] - spikes_in[(h + 3) % _NUM_IN]
                v_new = vh[h] + (agg - vh[h]) * _INV_TAU
                fired = v_new >= _THRESHOLD
                hs.append(jnp.where(fired, ones, zeros))
                vh[h] = jnp.where(fired, zeros, v_new)
            for o in range(_NUM_OUT):
                agg = (hs[2 * o] + hs[2 * o + 1]
                       - hs[(2 * o + 4) % _NUM_HID] - hs[(2 * o + 5) % _NUM_HID])
                v_new = vo[o] + (agg - vo[o]) * _INV_TAU
                fired = v_new >= _THRESHOLD
                vo[o] = jnp.where(fired, zeros, v_new)
                acc[o] = acc[o] + jnp.where(fired, active, zeros)
        for o in range(_NUM_OUT):
            ov[pl.ds(o * chunk + off, _L)] = acc[o]
        return tuple(rng)

    lax.fori_loop(0, num_groups, group_body, tuple(seeds0))

    out_copies = [
        pltpu.make_async_copy(
            ov.at[pl.ds(o * chunk, chunk)],
            out_hbm.at[o, pl.ds(base - sc_base, chunk)],
            sem,
        )
        for o in range(_NUM_OUT)
    ]
    for cp in out_copies:
        cp.start()
    for cp in out_copies:
        cp.wait()


def _tc_body(ns_ref, x_ref, out_ref):
    bt = x_ref.shape[1]
    pltpu.prng_seed(0x5CBA17)
    thr = (x_ref[...] * np.float32(1 << 24)).astype(jnp.int32)
    ns = ns_ref[0]
    zero = np.float32(0.0)
    one = np.float32(1.0)
    vh = jnp.zeros((_NUM_HID, bt), jnp.float32)
    vo = jnp.zeros((_NUM_HID, bt), jnp.float32)
    acc = jnp.zeros((_NUM_HID, bt), jnp.float32)
    for t in range(_STEPS):
        bits = pltpu.prng_random_bits((_NUM_IN, bt)).astype(jnp.uint32)
        u = (bits >> np.uint32(8)).astype(jnp.int32)
        spikes = jnp.where(u < thr, one, zero)
        aggh = spikes - jnp.concatenate([spikes[3:], spikes[:3]], axis=0)
        v_new = vh + (aggh - vh) * _INV_TAU
        firedh = v_new >= _THRESHOLD
        hsp = jnp.where(firedh, one, zero)
        vh = jnp.where(firedh, zero, v_new)
        q = hsp + jnp.concatenate([hsp[1:], hsp[:1]], axis=0)
        aggo = q - jnp.concatenate([q[4:], q[:4]], axis=0)
        v_new_o = vo + (aggo - vo) * _INV_TAU
        firedo = v_new_o >= _THRESHOLD
        vo = jnp.where(firedo, zero, v_new_o)
        active = jnp.where(t < ns, one, zero)
        acc = acc + jnp.where(firedo, active, zero)
    for o in range(_NUM_OUT):
        out_ref[o, :] = acc[2 * o, :]


def kernel(x, num_steps):
    batch = x.shape[0]
    num_workers = 32
    b_tc = (batch * 3) // 4
    b_sc = batch - b_tc
    assert b_sc % (num_workers * _L) == 0 and b_tc % 128 == 0
    chunk = b_sc // num_workers
    ns_arr = jnp.full((_L,), num_steps, dtype=jnp.int32)
    xt = x.T

    mesh = plsc.VectorSubcoreMesh(core_axis_name="c", subcore_axis_name="s")
    run_sc = pl.kernel(
        functools.partial(_sc_body, num_workers, chunk, b_tc),
        out_type=jax.ShapeDtypeStruct((_NUM_OUT, b_sc), jnp.float32),
        mesh=mesh,
        compiler_params=pltpu.CompilerParams(needs_layout_passes=False),
        scratch_types=[
            pltpu.VMEM((chunk * _NUM_IN,), jnp.float32),
            pltpu.VMEM((chunk * _NUM_OUT,), jnp.float32),
            pltpu.VMEM((_L,), jnp.int32),
            pltpu.SemaphoreType.DMA,
        ],
    )
    sc_out = run_sc(xt, ns_arr)

    tc_out = pl.pallas_call(
        _tc_body,
        out_shape=jax.ShapeDtypeStruct((_NUM_OUT, b_tc), jnp.float32),
        grid=(1,),
        in_specs=[
            pl.BlockSpec(memory_space=pltpu.SMEM),
            pl.BlockSpec((_NUM_IN, b_tc), lambda i: (0, 0)),
        ],
        out_specs=pl.BlockSpec((_NUM_OUT, b_tc), lambda i: (0, 0)),
    )(ns_arr, xt)

    out_t = jnp.concatenate([tc_out, sc_out], axis=1)
    return out_t.T


# speedup vs baseline: 9.7855x; 1.0582x over previous
import functools

import numpy as np
import jax
import jax.numpy as jnp
from jax import lax
from jax.experimental import pallas as pl
from jax.experimental.pallas import tpu as pltpu
from jax.experimental.pallas import tpu_sc as plsc

_BETA = 0.9
_TAU = 1.0 / (1.0 - _BETA)
_INV_TAU = np.float32(1.0 / _TAU)
_THRESHOLD = np.float32(1.0)
_NUM_IN = 8
_NUM_HID = 8
_NUM_OUT = 4
_STEPS = 16
_L = 16

_LCG_A = np.uint32(747796405)
_LCG_C = np.uint32(2891336453)
_SEED_M = np.uint32(2654435761)
_SEED_C = np.uint32(0x9E3779B9)


def _sc_body(ncores, chunk, sc_base, x_hbm, ns_hbm, out_hbm, xv, ov, nsv, sem):
    cid = lax.axis_index("c")
    sid = lax.axis_index("s")
    wid = sid * ncores + cid
    base = sc_base + wid * chunk

    copies = [
        pltpu.make_async_copy(
            x_hbm.at[c, pl.ds(base, chunk)],
            xv.at[pl.ds(c * chunk, chunk)],
            sem,
        )
        for c in range(_NUM_IN)
    ]
    for cp in copies:
        cp.start()
    pltpu.sync_copy(ns_hbm, nsv)
    for cp in copies:
        cp.wait()

    iota = lax.iota(jnp.int32, _L)
    ones = jnp.full((_L,), 1.0, jnp.float32)
    zeros = jnp.full((_L,), 0.0, jnp.float32)
    ns_i = nsv[...]

    actives = [
        jnp.where(jnp.full((_L,), t, jnp.int32) < ns_i, ones, zeros)
        for t in range(_STEPS)
    ]

    gidx = jnp.full((_L,), base, jnp.int32) + iota
    seeds0 = []
    for p in range(_NUM_IN // 2):
        s = (gidx * np.int32(_NUM_IN // 2) + np.int32(p)).astype(jnp.uint32)
        s = s * _SEED_M + _SEED_C
        s = s ^ (s >> np.uint32(16))
        s = s * _LCG_A + _LCG_C
        seeds0.append(s)

    num_groups = chunk // _L

    def group_body(g, rng):
        off = g * np.int32(_L)
        thr = [
            (xv[pl.ds(c * chunk + off, _L)] * np.float32(1 << 16))
            .astype(jnp.int32)
            for c in range(_NUM_IN)
        ]

        rng = list(rng)
        vh = [zeros] * _NUM_HID
        vo = [zeros] * _NUM_OUT
        acc = [zeros] * _NUM_OUT
        for t in range(_STEPS):
            active = actives[t]
            spikes_in = [None] * _NUM_IN
            for p in range(_NUM_IN // 2):
                s = rng[p] * _LCG_A + _LCG_C
                u_hi = plsc.bitcast(s >> np.uint32(16), jnp.int32)
                u_lo = plsc.bitcast(s & np.uint32(0xFFFF), jnp.int32)
                spikes_in[p] = jnp.where(u_hi < thr[p], ones, zeros)
                spikes_in[p + 4] = jnp.where(u_lo < thr[p + 4], ones, zeros)
                rng[p] = s
            hs = []
            for h in range(_NUM_HID):
                agg = spikes_in[h] - spikes_in[(h + 3) % _NUM_IN]
                v_new = vh[h] + (agg - vh[h]) * _INV_TAU
                fired = v_new >= _THRESHOLD
                hs.append(jnp.where(fired, ones, zeros))
                vh[h] = jnp.where(fired, zeros, v_new)
            for o in range(_NUM_OUT):
                agg = (hs[2 * o] + hs[2 * o + 1]
                       - hs[(2 * o + 4) % _NUM_HID] - hs[(2 * o + 5) % _NUM_HID])
                v_new = vo[o] + (agg - vo[o]) * _INV_TAU
                fired = v_new >= _THRESHOLD
                vo[o] = jnp.where(fired, zeros, v_new)
                acc[o] = acc[o] + jnp.where(fired, active, zeros)
        for o in range(_NUM_OUT):
            ov[pl.ds(o * chunk + off, _L)] = acc[o]
        return tuple(rng)

    lax.fori_loop(0, num_groups, group_body, tuple(seeds0))

    out_copies = [
        pltpu.make_async_copy(
            ov.at[pl.ds(o * chunk, chunk)],
            out_hbm.at[o, pl.ds(base - sc_base, chunk)],
            sem,
        )
        for o in range(_NUM_OUT)
    ]
    for cp in out_copies:
        cp.start()
    for cp in out_copies:
        cp.wait()


def _tc_body(ns_ref, x_ref, out_ref):
    bt = x_ref.shape[1]
    pltpu.prng_seed(0x5CBA17)
    thr = (x_ref[...] * np.float32(1 << 24)).astype(jnp.int32)
    ns = ns_ref[0]
    zero = np.float32(0.0)
    one = np.float32(1.0)
    vh = jnp.zeros((_NUM_HID, bt), jnp.float32)
    vo = jnp.zeros((_NUM_HID, bt), jnp.float32)
    acc = jnp.zeros((_NUM_HID, bt), jnp.float32)
    for t in range(_STEPS):
        bits = pltpu.prng_random_bits((_NUM_IN, bt)).astype(jnp.uint32)
        u = (bits >> np.uint32(8)).astype(jnp.int32)
        spikes = jnp.where(u < thr, one, zero)
        aggh = spikes - jnp.concatenate([spikes[3:], spikes[:3]], axis=0)
        v_new = vh + (aggh - vh) * _INV_TAU
        firedh = v_new >= _THRESHOLD
        hsp = jnp.where(firedh, one, zero)
        vh = jnp.where(firedh, zero, v_new)
        q = hsp + jnp.concatenate([hsp[1:], hsp[:1]], axis=0)
        aggo = q - jnp.concatenate([q[4:], q[:4]], axis=0)
        v_new_o = vo + (aggo - vo) * _INV_TAU
        firedo = v_new_o >= _THRESHOLD
        vo = jnp.where(firedo, zero, v_new_o)
        active = jnp.where(t < ns, one, zero)
        acc = acc + jnp.where(firedo, active, zero)
    for o in range(_NUM_OUT):
        out_ref[o, :] = acc[2 * o, :]


def kernel(x, num_steps):
    batch = x.shape[0]
    num_cores = 1
    num_workers = num_cores * 16
    b_tc = (batch * 7) // 8
    b_sc = batch - b_tc
    assert b_sc % (num_workers * _L) == 0 and b_tc % 128 == 0
    chunk = b_sc // num_workers
    ns_arr = jnp.full((_L,), num_steps, dtype=jnp.int32)
    xt = x.T

    mesh = plsc.VectorSubcoreMesh(
        core_axis_name="c", subcore_axis_name="s", num_cores=num_cores)
    run_sc = pl.kernel(
        functools.partial(_sc_body, num_cores, chunk, b_tc),
        out_type=jax.ShapeDtypeStruct((_NUM_OUT, b_sc), jnp.float32),
        mesh=mesh,
        compiler_params=pltpu.CompilerParams(needs_layout_passes=False),
        scratch_types=[
            pltpu.VMEM((chunk * _NUM_IN,), jnp.float32),
            pltpu.VMEM((chunk * _NUM_OUT,), jnp.float32),
            pltpu.VMEM((_L,), jnp.int32),
            pltpu.SemaphoreType.DMA,
        ],
    )
    sc_out = run_sc(xt, ns_arr)

    tc_out = pl.pallas_call(
        _tc_body,
        out_shape=jax.ShapeDtypeStruct((_NUM_OUT, b_tc), jnp.float32),
        grid=(1,),
        in_specs=[
            pl.BlockSpec(memory_space=pltpu.SMEM),
            pl.BlockSpec((_NUM_IN, b_tc), lambda i: (0, 0)),
        ],
        out_specs=pl.BlockSpec((_NUM_OUT, b_tc), lambda i: (0, 0)),
    )(ns_arr, xt)

    out_t = jnp.concatenate([tc_out, sc_out], axis=1)
    return out_t.T


# speedup vs baseline: 9.8852x; 1.0102x over previous
import functools

import numpy as np
import jax
import jax.numpy as jnp
from jax import lax
from jax.experimental import pallas as pl
from jax.experimental.pallas import tpu as pltpu
from jax.experimental.pallas import tpu_sc as plsc

_BETA = 0.9
_TAU = 1.0 / (1.0 - _BETA)
_INV_TAU = np.float32(1.0 / _TAU)
_THRESHOLD = np.float32(1.0)
_NUM_IN = 8
_NUM_HID = 8
_NUM_OUT = 4
_STEPS = 16
_L = 16

_LCG_A = np.uint32(747796405)
_LCG_C = np.uint32(2891336453)
_SEED_M = np.uint32(2654435761)
_SEED_C = np.uint32(0x9E3779B9)


def _sc_body(ncores, chunk, sc_base, x_hbm, ns_hbm, out_hbm, xv, ov, nsv, sem):
    cid = lax.axis_index("c")
    sid = lax.axis_index("s")
    wid = sid * ncores + cid
    base = sc_base + wid * chunk

    copies = [
        pltpu.make_async_copy(
            x_hbm.at[c, pl.ds(base, chunk)],
            xv.at[pl.ds(c * chunk, chunk)],
            sem,
        )
        for c in range(_NUM_IN)
    ]
    for cp in copies:
        cp.start()
    pltpu.sync_copy(ns_hbm, nsv)
    for cp in copies:
        cp.wait()

    iota = lax.iota(jnp.int32, _L)
    ones = jnp.full((_L,), 1.0, jnp.float32)
    zeros = jnp.full((_L,), 0.0, jnp.float32)
    ns_i = nsv[...]

    actives = [
        jnp.where(jnp.full((_L,), t, jnp.int32) < ns_i, ones, zeros)
        for t in range(_STEPS)
    ]

    gidx = jnp.full((_L,), base, jnp.int32) + iota
    seeds0 = []
    for p in range(_NUM_IN // 2):
        s = (gidx * np.int32(_NUM_IN // 2) + np.int32(p)).astype(jnp.uint32)
        s = s * _SEED_M + _SEED_C
        s = s ^ (s >> np.uint32(16))
        s = s * _LCG_A + _LCG_C
        seeds0.append(s)

    num_groups = chunk // _L

    def group_body(g, rng):
        off = g * np.int32(_L)
        thr = [
            (xv[pl.ds(c * chunk + off, _L)] * np.float32(1 << 16))
            .astype(jnp.int32)
            for c in range(_NUM_IN)
        ]

        rng = list(rng)
        vh = [zeros] * _NUM_HID
        vo = [zeros] * _NUM_OUT
        acc = [zeros] * _NUM_OUT
        for t in range(_STEPS):
            active = actives[t]
            spikes_in = [None] * _NUM_IN
            for p in range(_NUM_IN // 2):
                s = rng[p] * _LCG_A + _LCG_C
                u_hi = plsc.bitcast(s >> np.uint32(16), jnp.int32)
                u_lo = plsc.bitcast(s & np.uint32(0xFFFF), jnp.int32)
                spikes_in[p] = jnp.where(u_hi < thr[p], ones, zeros)
                spikes_in[p + 4] = jnp.where(u_lo < thr[p + 4], ones, zeros)
                rng[p] = s
            hs = []
            for h in range(_NUM_HID):
                agg = spikes_in[h] - spikes_in[(h + 3) % _NUM_IN]
                v_new = vh[h] + (agg - vh[h]) * _INV_TAU
                fired = v_new >= _THRESHOLD
                hs.append(jnp.where(fired, ones, zeros))
                vh[h] = jnp.where(fired, zeros, v_new)
            q = [hs[2 * e] + hs[2 * e + 1] for e in range(_NUM_OUT)]
            for o in range(_NUM_OUT):
                agg = q[o] - q[(o + 2) % _NUM_OUT]
                v_new = vo[o] + (agg - vo[o]) * _INV_TAU
                fired = v_new >= _THRESHOLD
                vo[o] = jnp.where(fired, zeros, v_new)
                acc[o] = acc[o] + jnp.where(fired, active, zeros)
        for o in range(_NUM_OUT):
            ov[pl.ds(o * chunk + off, _L)] = acc[o]
        return tuple(rng)

    lax.fori_loop(0, num_groups, group_body, tuple(seeds0))

    out_copies = [
        pltpu.make_async_copy(
            ov.at[pl.ds(o * chunk, chunk)],
            out_hbm.at[o, pl.ds(base - sc_base, chunk)],
            sem,
        )
        for o in range(_NUM_OUT)
    ]
    for cp in out_copies:
        cp.start()
    for cp in out_copies:
        cp.wait()


def _tc_body(ns_ref, x_ref, out_ref):
    bt = x_ref.shape[1]
    pltpu.prng_seed(0x5CBA17)
    thr = (x_ref[...] * np.float32(1 << 24)).astype(jnp.int32)
    ns = ns_ref[0]
    zero = np.float32(0.0)
    one = np.float32(1.0)
    vh = jnp.zeros((_NUM_HID, bt), jnp.float32)
    vo = jnp.zeros((_NUM_HID, bt), jnp.float32)
    acc = jnp.zeros((_NUM_HID, bt), jnp.float32)
    for t in range(_STEPS):
        bits = pltpu.prng_random_bits((_NUM_IN, bt)).astype(jnp.uint32)
        u = (bits >> np.uint32(8)).astype(jnp.int32)
        spikes = jnp.where(u < thr, one, zero)
        aggh = spikes - jnp.concatenate([spikes[3:], spikes[:3]], axis=0)
        v_new = vh + (aggh - vh) * _INV_TAU
        firedh = v_new >= _THRESHOLD
        hsp = jnp.where(firedh, one, zero)
        vh = jnp.where(firedh, zero, v_new)
        q = hsp + jnp.concatenate([hsp[1:], hsp[:1]], axis=0)
        aggo = q - jnp.concatenate([q[4:], q[:4]], axis=0)
        v_new_o = vo + (aggo - vo) * _INV_TAU
        firedo = v_new_o >= _THRESHOLD
        vo = jnp.where(firedo, zero, v_new_o)
        active = jnp.where(t < ns, one, zero)
        acc = acc + jnp.where(firedo, active, zero)
    for o in range(_NUM_OUT):
        out_ref[o, :] = acc[2 * o, :]


def kernel(x, num_steps):
    batch = x.shape[0]
    num_cores = 1
    num_workers = num_cores * 16
    b_tc = (batch * 7) // 8
    b_sc = batch - b_tc
    assert b_sc % (num_workers * _L) == 0 and b_tc % 128 == 0
    chunk = b_sc // num_workers
    ns_arr = jnp.full((_L,), num_steps, dtype=jnp.int32)
    xt = x.T

    mesh = plsc.VectorSubcoreMesh(
        core_axis_name="c", subcore_axis_name="s", num_cores=num_cores)
    run_sc = pl.kernel(
        functools.partial(_sc_body, num_cores, chunk, b_tc),
        out_type=jax.ShapeDtypeStruct((_NUM_OUT, b_sc), jnp.float32),
        mesh=mesh,
        compiler_params=pltpu.CompilerParams(needs_layout_passes=False),
        scratch_types=[
            pltpu.VMEM((chunk * _NUM_IN,), jnp.float32),
            pltpu.VMEM((chunk * _NUM_OUT,), jnp.float32),
            pltpu.VMEM((_L,), jnp.int32),
            pltpu.SemaphoreType.DMA,
        ],
    )
    sc_out = run_sc(xt, ns_arr)

    tc_out = pl.pallas_call(
        _tc_body,
        out_shape=jax.ShapeDtypeStruct((_NUM_OUT, b_tc), jnp.float32),
        grid=(1,),
        in_specs=[
            pl.BlockSpec(memory_space=pltpu.SMEM),
            pl.BlockSpec((_NUM_IN, b_tc), lambda i: (0, 0)),
        ],
        out_specs=pl.BlockSpec((_NUM_OUT, b_tc), lambda i: (0, 0)),
    )(ns_arr, xt)

    out_t = jnp.concatenate([tc_out, sc_out], axis=1)
    return out_t.T
